# Initial kernel scaffold; baseline (speedup 1.0000x reference)
import jax
import jax.numpy as jnp
from jax.experimental import pallas as pl


def kernel(x, edge_index, Wq, bq, Wk, bk, Wv, bv):
    raise NotImplementedError("write your pallas kernel here")


# speedup vs baseline: 54.7761x; 54.7761x over previous
import functools

import jax
import jax.numpy as jnp
from jax import lax
from jax.experimental import pallas as pl
from jax.experimental.pallas import tpu as pltpu
from jax.experimental.pallas import tpu_sc as plsc

LANES = 16
NC = 2
NS = 16


def _qk_body(x_ref, wqk_ref, bqk_ref, q_ref, k_ref):
    qk = jnp.dot(x_ref[...], wqk_ref[...],
                 preferred_element_type=jnp.float32) + bqk_ref[...]
    q_ref[...] = qk[:, 0:1]
    k_ref[...] = qk[:, 1:2]


def _compute_qk(x, Wqk, bqk):
    n, d = x.shape
    grid = 10
    blk = n // grid
    return pl.pallas_call(
        _qk_body,
        grid=(grid,),
        in_specs=[
            pl.BlockSpec((blk, d), lambda i: (i, 0)),
            pl.BlockSpec((d, 2), lambda i: (0, 0)),
            pl.BlockSpec((1, 2), lambda i: (0, 0)),
        ],
        out_specs=[
            pl.BlockSpec((blk, 1), lambda i: (i, 0)),
            pl.BlockSpec((blk, 1), lambda i: (i, 0)),
        ],
        out_shape=[
            jax.ShapeDtypeStruct((n, 1), jnp.float32),
            jax.ShapeDtypeStruct((n, 1), jnp.float32),
        ],
    )(x, Wqk, bqk)


def _make_sc_edge(n, e):
    epw = e // (NC * NS)
    vpi = epw // LANES
    nvec = n // LANES
    stripe = 640
    last_stripe = n - stripe * (NS - 1)

    mesh = plsc.VectorSubcoreMesh(core_axis_name="c", subcore_axis_name="s")

    @functools.partial(
        pl.kernel,
        out_type=(
            jax.ShapeDtypeStruct((NC * n,), jnp.float32),
            jax.ShapeDtypeStruct((NC * 2 * LANES,), jnp.float32),
        ),
        mesh=mesh,
        compiler_params=pltpu.CompilerParams(needs_layout_passes=False),
        scratch_types=[
            pltpu.VMEM((n,), jnp.float32),
            pltpu.VMEM((n,), jnp.float32),
            pltpu.VMEM((epw,), jnp.int32),
            pltpu.VMEM((epw,), jnp.int32),
            pltpu.VMEM((epw,), jnp.float32),
            pltpu.VMEM((n,), jnp.float32),
            pltpu.VMEM((LANES,), jnp.float32),
            pltpu.VMEM((NS * LANES,), jnp.float32),
            pltpu.VMEM((NS * stripe,), jnp.float32),
            pltpu.VMEM((stripe,), jnp.float32),
            pltpu.VMEM((2 * LANES,), jnp.float32),
            pltpu.VMEM_SHARED((NS * LANES,), jnp.float32),
            pltpu.VMEM_SHARED((NS * LANES,), jnp.float32),
            pltpu.VMEM_SHARED((NS * n,), jnp.float32),
        ],
    )
    def sc_edge(q_hbm, k_hbm, row_hbm, col_hbm, aw_hbm, stats_hbm,
                q_v, k_v, row_v, col_v, att_v, aw_v,
                red16_v, red_v, stripe_v, out_v, stat_v,
                max_sh, sum_sh, aw_sh):
        c = lax.axis_index("c")
        s = lax.axis_index("s")
        ebase = (c * NS + s) * epw

        pltpu.sync_copy(q_hbm, q_v)
        pltpu.sync_copy(k_hbm, k_v)
        pltpu.sync_copy(row_hbm.at[pl.ds(ebase, epw)], row_v)
        pltpu.sync_copy(col_hbm.at[pl.ds(ebase, epw)], col_v)

        zeros = jnp.zeros((LANES,), jnp.float32)

        def zero_body(i, carry):
            aw_v[pl.ds(i * LANES, LANES)] = zeros
            return carry

        lax.fori_loop(0, nvec, zero_body, 0)

        def pass1(i, m):
            ridx = row_v[pl.ds(i * LANES, LANES)]
            cidx = col_v[pl.ds(i * LANES, LANES)]
            a = plsc.load_gather(q_v, [ridx]) * plsc.load_gather(k_v, [cidx])
            a = jnp.where(a >= 0.0, a, a * 0.2)
            att_v[pl.ds(i * LANES, LANES)] = a
            return jnp.maximum(m, a)

        mloc = lax.fori_loop(0, vpi, pass1,
                             jnp.full((LANES,), -jnp.inf, jnp.float32))

        red16_v[...] = mloc
        pltpu.sync_copy(red16_v, max_sh.at[pl.ds(s * LANES, LANES)])
        plsc.subcore_barrier()
        pltpu.sync_copy(max_sh, red_v)
        gm = red_v[pl.ds(0, LANES)]
        for t in range(1, NS):
            gm = jnp.maximum(gm, red_v[pl.ds(t * LANES, LANES)])
        m_vec = jnp.full((LANES,), jnp.max(gm), jnp.float32)

        def pass2(i, ssum):
            ridx = row_v[pl.ds(i * LANES, LANES)]
            p = jnp.exp(att_v[pl.ds(i * LANES, LANES)] - m_vec)
            plsc.addupdate_scatter(aw_v, [ridx], p)
            return ssum + p

        ssum = lax.fori_loop(0, vpi, pass2, jnp.zeros((LANES,), jnp.float32))

        red16_v[...] = ssum
        pltpu.sync_copy(red16_v, sum_sh.at[pl.ds(s * LANES, LANES)])
        pltpu.sync_copy(aw_v, aw_sh.at[pl.ds(s * n, n)])
        plsc.subcore_barrier()

        def reduce_stripe(base, size):
            for t in range(NS):
                pltpu.sync_copy(aw_sh.at[pl.ds(t * n + base, size)],
                                stripe_v.at[pl.ds(t * stripe, size)])

            def rbody(j, carry):
                acc = stripe_v[pl.ds(j * LANES, LANES)]
                for t in range(1, NS):
                    acc = acc + stripe_v[pl.ds(t * stripe + j * LANES, LANES)]
                out_v[pl.ds(j * LANES, LANES)] = acc
                return carry

            lax.fori_loop(0, size // LANES, rbody, 0)
            pltpu.sync_copy(out_v.at[pl.ds(0, size)],
                            aw_hbm.at[pl.ds(c * n + base, size)])

        @pl.when(s < NS - 1)
        def _():
            reduce_stripe(s * stripe, stripe)

        @pl.when(s == NS - 1)
        def _():
            reduce_stripe((NS - 1) * stripe, last_stripe)

        @pl.when(s == 0)
        def _():
            pltpu.sync_copy(sum_sh, red_v)
            sv = red_v[pl.ds(0, LANES)]
            for t in range(1, NS):
                sv = sv + red_v[pl.ds(t * LANES, LANES)]
            stat_v[pl.ds(0, LANES)] = m_vec
            stat_v[pl.ds(LANES, LANES)] = sv
            pltpu.sync_copy(stat_v, stats_hbm.at[pl.ds(c * 2 * LANES, 2 * LANES)])

    return sc_edge


def _fin_body(stats_ref, awp_ref, x_ref, wv_ref, bv_ref, aw_ref, ge_ref):
    stats = stats_ref[...]
    m = stats[:, 0:1]
    w = jnp.exp(m - jnp.max(m))
    s = jnp.sum(stats[:, LANES:2 * LANES], axis=1, keepdims=True)
    z = jnp.sum(w * s)
    awb = jnp.sum(w * awp_ref[...], axis=0, keepdims=True) / z
    aw_ref[...] = awb
    part = jnp.dot(awb, x_ref[...], preferred_element_type=jnp.float32)
    ge_ref[...] = jnp.dot(part, wv_ref[...],
                          preferred_element_type=jnp.float32) + bv_ref[...]


def _finish(stats, aw_part, x, Wv, bv2):
    n, d = x.shape
    return pl.pallas_call(
        _fin_body,
        out_shape=[
            jax.ShapeDtypeStruct((1, n), jnp.float32),
            jax.ShapeDtypeStruct((1, d), jnp.float32),
        ],
    )(stats, aw_part, x, Wv, bv2)


def kernel(x, edge_index, Wq, bq, Wk, bk, Wv, bv):
    n, d = x.shape
    e = edge_index.shape[1]
    Wqk = jnp.concatenate([Wq, Wk], axis=1)
    bqk = jnp.concatenate([bq, bk]).reshape(1, 2)
    q2, k2 = _compute_qk(x, Wqk, bqk)
    aw_part, stats = _make_sc_edge(n, e)(
        q2.reshape(n), k2.reshape(n), edge_index[0], edge_index[1])
    aw2, ge2 = _finish(stats.reshape(NC, 2 * LANES), aw_part.reshape(NC, n),
                       x, Wv, bv.reshape(1, d))
    return (ge2.reshape(d), aw2.reshape(n))


# speedup vs baseline: 61.5492x; 1.1237x over previous
import functools

import jax
import jax.numpy as jnp
from jax import lax
from jax.experimental import pallas as pl
from jax.experimental.pallas import tpu as pltpu
from jax.experimental.pallas import tpu_sc as plsc

LANES = 16
NC = 2
NS = 16


def _qk_body(x_ref, wqk_ref, bqk_ref, q_ref, k_ref):
    qk = jnp.dot(x_ref[...], wqk_ref[...],
                 preferred_element_type=jnp.float32) + bqk_ref[...]
    q_ref[...] = qk[:, 0:1]
    k_ref[...] = qk[:, 1:2]


def _compute_qk(x, Wqk, bqk):
    n, d = x.shape
    grid = 10
    blk = n // grid
    return pl.pallas_call(
        _qk_body,
        grid=(grid,),
        in_specs=[
            pl.BlockSpec((blk, d), lambda i: (i, 0)),
            pl.BlockSpec((d, 2), lambda i: (0, 0)),
            pl.BlockSpec((1, 2), lambda i: (0, 0)),
        ],
        out_specs=[
            pl.BlockSpec((blk, 1), lambda i: (i, 0)),
            pl.BlockSpec((blk, 1), lambda i: (i, 0)),
        ],
        out_shape=[
            jax.ShapeDtypeStruct((n, 1), jnp.float32),
            jax.ShapeDtypeStruct((n, 1), jnp.float32),
        ],
    )(x, Wqk, bqk)


def _make_sc_edge(n, e):
    epw = e // (NC * NS)
    vpi = epw // LANES
    nvec = n // LANES
    stripe = 640
    last_stripe = n - stripe * (NS - 1)

    mesh = plsc.VectorSubcoreMesh(core_axis_name="c", subcore_axis_name="s")

    @functools.partial(
        pl.kernel,
        out_type=(
            jax.ShapeDtypeStruct((NC * n,), jnp.float32),
            jax.ShapeDtypeStruct((NC * 2 * LANES,), jnp.float32),
        ),
        mesh=mesh,
        compiler_params=pltpu.CompilerParams(needs_layout_passes=False),
        scratch_types=[
            pltpu.VMEM((n,), jnp.float32),
            pltpu.VMEM((n,), jnp.float32),
            pltpu.VMEM((epw,), jnp.int32),
            pltpu.VMEM((epw,), jnp.int32),
            pltpu.VMEM((epw,), jnp.float32),
            pltpu.VMEM((n,), jnp.float32),
            pltpu.VMEM((LANES,), jnp.float32),
            pltpu.VMEM((NS * LANES,), jnp.float32),
            pltpu.VMEM((NS * stripe,), jnp.float32),
            pltpu.VMEM((stripe,), jnp.float32),
            pltpu.VMEM((2 * LANES,), jnp.float32),
            pltpu.VMEM_SHARED((NS * LANES,), jnp.float32),
            pltpu.VMEM_SHARED((NS * LANES,), jnp.float32),
            pltpu.VMEM_SHARED((NS * n,), jnp.float32),
        ],
    )
    def sc_edge(q_hbm, k_hbm, edge_hbm, aw_hbm, stats_hbm,
                q_v, k_v, row_v, col_v, att_v, aw_v,
                red16_v, red_v, stripe_v, out_v, stat_v,
                max_sh, sum_sh, aw_sh):
        c = lax.axis_index("c")
        s = lax.axis_index("s")
        ebase = (c * NS + s) * epw

        pltpu.sync_copy(q_hbm, q_v)
        pltpu.sync_copy(k_hbm, k_v)
        pltpu.sync_copy(edge_hbm.at[pl.ds(ebase, epw)], row_v)
        pltpu.sync_copy(edge_hbm.at[pl.ds(e + ebase, epw)], col_v)

        zeros = jnp.zeros((LANES,), jnp.float32)

        def zero_body(i, carry):
            aw_v[pl.ds(i * LANES, LANES)] = zeros
            return carry

        lax.fori_loop(0, nvec, zero_body, 0)

        def pass1(i, m):
            ridx = row_v[pl.ds(i * LANES, LANES)]
            cidx = col_v[pl.ds(i * LANES, LANES)]
            a = plsc.load_gather(q_v, [ridx]) * plsc.load_gather(k_v, [cidx])
            a = jnp.where(a >= 0.0, a, a * 0.2)
            att_v[pl.ds(i * LANES, LANES)] = a
            return jnp.maximum(m, a)

        mloc = lax.fori_loop(0, vpi, pass1,
                             jnp.full((LANES,), -jnp.inf, jnp.float32))

        red16_v[...] = mloc
        pltpu.sync_copy(red16_v, max_sh.at[pl.ds(s * LANES, LANES)])
        plsc.subcore_barrier()
        pltpu.sync_copy(max_sh, red_v)
        gm = red_v[pl.ds(0, LANES)]
        for t in range(1, NS):
            gm = jnp.maximum(gm, red_v[pl.ds(t * LANES, LANES)])
        m_vec = jnp.full((LANES,), jnp.max(gm), jnp.float32)

        def pass2(i, ssum):
            ridx = row_v[pl.ds(i * LANES, LANES)]
            p = jnp.exp(att_v[pl.ds(i * LANES, LANES)] - m_vec)
            plsc.addupdate_scatter(aw_v, [ridx], p)
            return ssum + p

        ssum = lax.fori_loop(0, vpi, pass2, jnp.zeros((LANES,), jnp.float32))

        red16_v[...] = ssum
        pltpu.sync_copy(red16_v, sum_sh.at[pl.ds(s * LANES, LANES)])
        pltpu.sync_copy(aw_v, aw_sh.at[pl.ds(s * n, n)])
        plsc.subcore_barrier()

        def reduce_stripe(base, size):
            for t in range(NS):
                pltpu.sync_copy(aw_sh.at[pl.ds(t * n + base, size)],
                                stripe_v.at[pl.ds(t * stripe, size)])

            def rbody(j, carry):
                acc = stripe_v[pl.ds(j * LANES, LANES)]
                for t in range(1, NS):
                    acc = acc + stripe_v[pl.ds(t * stripe + j * LANES, LANES)]
                out_v[pl.ds(j * LANES, LANES)] = acc
                return carry

            lax.fori_loop(0, size // LANES, rbody, 0)
            pltpu.sync_copy(out_v.at[pl.ds(0, size)],
                            aw_hbm.at[pl.ds(c * n + base, size)])

        @pl.when(s < NS - 1)
        def _():
            reduce_stripe(s * stripe, stripe)

        @pl.when(s == NS - 1)
        def _():
            reduce_stripe((NS - 1) * stripe, last_stripe)

        @pl.when(s == 0)
        def _():
            pltpu.sync_copy(sum_sh, red_v)
            sv = red_v[pl.ds(0, LANES)]
            for t in range(1, NS):
                sv = sv + red_v[pl.ds(t * LANES, LANES)]
            stat_v[pl.ds(0, LANES)] = m_vec
            stat_v[pl.ds(LANES, LANES)] = sv
            pltpu.sync_copy(stat_v, stats_hbm.at[pl.ds(c * 2 * LANES, 2 * LANES)])

    return sc_edge


def _fin_body(stats_ref, awp_ref, x_ref, wv_ref, bv_ref, aw_ref, ge_ref):
    stats = stats_ref[...]
    m = stats[:, 0:1]
    w = jnp.exp(m - jnp.max(m))
    s = jnp.sum(stats[:, LANES:2 * LANES], axis=1, keepdims=True)
    z = jnp.sum(w * s)
    awb = jnp.sum(w * awp_ref[...], axis=0, keepdims=True) / z
    aw_ref[...] = awb
    part = jnp.dot(awb, x_ref[...], preferred_element_type=jnp.float32)
    ge_ref[...] = jnp.dot(part, wv_ref[...],
                          preferred_element_type=jnp.float32) + bv_ref[...]


def _finish(stats, aw_part, x, Wv, bv2):
    n, d = x.shape
    return pl.pallas_call(
        _fin_body,
        out_shape=[
            jax.ShapeDtypeStruct((1, n), jnp.float32),
            jax.ShapeDtypeStruct((1, d), jnp.float32),
        ],
    )(stats, aw_part, x, Wv, bv2)


def kernel(x, edge_index, Wq, bq, Wk, bk, Wv, bv):
    n, d = x.shape
    e = edge_index.shape[1]
    Wqk = jnp.concatenate([Wq, Wk], axis=1)
    bqk = jnp.concatenate([bq, bk]).reshape(1, 2)
    q2, k2 = _compute_qk(x, Wqk, bqk)
    aw_part, stats = _make_sc_edge(n, e)(
        q2.reshape(n), k2.reshape(n), edge_index.reshape(2 * e))
    aw2, ge2 = _finish(stats.reshape(NC, 2 * LANES), aw_part.reshape(NC, n),
                       x, Wv, bv.reshape(1, d))
    return (ge2.reshape(d), aw2.reshape(n))


# speedup vs baseline: 75.6487x; 1.2291x over previous
import functools

import jax
import jax.numpy as jnp
from jax import lax
from jax.experimental import pallas as pl
from jax.experimental.pallas import tpu as pltpu
from jax.experimental.pallas import tpu_sc as plsc

LANES = 16
NC = 2
NS = 16


def _qk_body(x_ref, wqk_ref, bqk_ref, q_ref, k_ref):
    qk = jnp.dot(x_ref[...], wqk_ref[...],
                 preferred_element_type=jnp.float32) + bqk_ref[...]
    q_ref[...] = qk[:, 0:1]
    k_ref[...] = qk[:, 1:2]


def _compute_qk(x, Wqk, bqk):
    n, d = x.shape
    grid = 10
    blk = n // grid
    return pl.pallas_call(
        _qk_body,
        grid=(grid,),
        in_specs=[
            pl.BlockSpec((blk, d), lambda i: (i, 0)),
            pl.BlockSpec((d, 2), lambda i: (0, 0)),
            pl.BlockSpec((1, 2), lambda i: (0, 0)),
        ],
        out_specs=[
            pl.BlockSpec((blk, 1), lambda i: (i, 0)),
            pl.BlockSpec((blk, 1), lambda i: (i, 0)),
        ],
        out_shape=[
            jax.ShapeDtypeStruct((n, 1), jnp.float32),
            jax.ShapeDtypeStruct((n, 1), jnp.float32),
        ],
    )(x, Wqk, bqk)


def _make_sc_edge(n, e):
    epw = e // (NC * NS)
    vpi = epw // LANES
    nvec = n // LANES
    stripe = 640
    last_stripe = n - stripe * (NS - 1)

    mesh = plsc.VectorSubcoreMesh(core_axis_name="c", subcore_axis_name="s")

    @functools.partial(
        pl.kernel,
        out_type=(
            jax.ShapeDtypeStruct((NC * n,), jnp.float32),
            jax.ShapeDtypeStruct((NC * 2 * LANES,), jnp.float32),
        ),
        mesh=mesh,
        compiler_params=pltpu.CompilerParams(needs_layout_passes=False),
        scratch_types=[
            pltpu.VMEM((n,), jnp.float32),
            pltpu.VMEM((n,), jnp.float32),
            pltpu.VMEM((epw,), jnp.int32),
            pltpu.VMEM((epw,), jnp.int32),
            pltpu.VMEM((epw,), jnp.float32),
            pltpu.VMEM((n,), jnp.float32),
            pltpu.VMEM((LANES,), jnp.float32),
            pltpu.VMEM((NS * LANES,), jnp.float32),
            pltpu.VMEM((NS * stripe,), jnp.float32),
            pltpu.VMEM((stripe,), jnp.float32),
            pltpu.VMEM((2 * LANES,), jnp.float32),
            pltpu.VMEM_SHARED((NS * LANES,), jnp.float32),
            pltpu.VMEM_SHARED((NS * LANES,), jnp.float32),
            pltpu.VMEM_SHARED((NS * n,), jnp.float32),
            pltpu.SemaphoreType.DMA,
        ],
    )
    def sc_edge(q_hbm, k_hbm, edge_hbm, aw_hbm, stats_hbm,
                q_v, k_v, row_v, col_v, att_v, aw_v,
                red16_v, red_v, stripe_v, out_v, stat_v,
                max_sh, sum_sh, aw_sh, sem):
        c = lax.axis_index("c")
        s = lax.axis_index("s")
        ebase = (c * NS + s) * epw

        cps = [pltpu.async_copy(q_hbm, q_v, sem),
               pltpu.async_copy(k_hbm, k_v, sem),
               pltpu.async_copy(edge_hbm.at[pl.ds(ebase, epw)], row_v, sem),
               pltpu.async_copy(edge_hbm.at[pl.ds(e + ebase, epw)], col_v, sem)]

        zeros = jnp.zeros((LANES,), jnp.float32)

        @plsc.parallel_loop(0, nvec * LANES, LANES, unroll=8)
        def _(i):
            aw_v[pl.ds(i, LANES)] = zeros

        for cp in cps:
            cp.wait()

        @plsc.parallel_loop(0, vpi * LANES, LANES, unroll=4,
                            carry=jnp.full((LANES,), -jnp.inf, jnp.float32))
        def mloc(i, m):
            ridx = row_v[pl.ds(i, LANES)]
            cidx = col_v[pl.ds(i, LANES)]
            a = plsc.load_gather(q_v, [ridx]) * plsc.load_gather(k_v, [cidx])
            a = jnp.where(a >= 0.0, a, a * 0.2)
            att_v[pl.ds(i, LANES)] = a
            return jnp.maximum(m, a)

        red16_v[...] = mloc
        pltpu.sync_copy(red16_v, max_sh.at[pl.ds(s * LANES, LANES)])
        plsc.subcore_barrier()
        pltpu.sync_copy(max_sh, red_v)
        gm = red_v[pl.ds(0, LANES)]
        for t in range(1, NS):
            gm = jnp.maximum(gm, red_v[pl.ds(t * LANES, LANES)])
        m_vec = jnp.full((LANES,), jnp.max(gm), jnp.float32)

        @plsc.parallel_loop(0, vpi * LANES, LANES, unroll=4,
                            carry=jnp.zeros((LANES,), jnp.float32))
        def ssum(i, acc):
            ridx = row_v[pl.ds(i, LANES)]
            p = jnp.exp(att_v[pl.ds(i, LANES)] - m_vec)
            plsc.addupdate_scatter(aw_v, [ridx], p)
            return acc + p

        red16_v[...] = ssum
        pltpu.sync_copy(red16_v, sum_sh.at[pl.ds(s * LANES, LANES)])
        pltpu.sync_copy(aw_v, aw_sh.at[pl.ds(s * n, n)])
        plsc.subcore_barrier()

        def reduce_stripe(base, size):
            scps = [pltpu.async_copy(aw_sh.at[pl.ds(t * n + base, size)],
                                     stripe_v.at[pl.ds(t * stripe, size)], sem)
                    for t in range(NS)]
            for cp in scps:
                cp.wait()

            @plsc.parallel_loop(0, size, LANES, unroll=2)
            def _(j):
                acc = stripe_v[pl.ds(j, LANES)]
                for t in range(1, NS):
                    acc = acc + stripe_v[pl.ds(t * stripe + j, LANES)]
                out_v[pl.ds(j, LANES)] = acc

            pltpu.sync_copy(out_v.at[pl.ds(0, size)],
                            aw_hbm.at[pl.ds(c * n + base, size)])

        @pl.when(s < NS - 1)
        def _():
            reduce_stripe(s * stripe, stripe)

        @pl.when(s == NS - 1)
        def _():
            reduce_stripe((NS - 1) * stripe, last_stripe)

        @pl.when(s == 0)
        def _():
            pltpu.sync_copy(sum_sh, red_v)
            sv = red_v[pl.ds(0, LANES)]
            for t in range(1, NS):
                sv = sv + red_v[pl.ds(t * LANES, LANES)]
            stat_v[pl.ds(0, LANES)] = m_vec
            stat_v[pl.ds(LANES, LANES)] = sv
            pltpu.sync_copy(stat_v, stats_hbm.at[pl.ds(c * 2 * LANES, 2 * LANES)])

    return sc_edge


def _fin_body(stats_ref, awp_ref, x_ref, wv_ref, bv_ref, aw_ref, ge_ref):
    stats = stats_ref[...]
    m = stats[:, 0:1]
    w = jnp.exp(m - jnp.max(m))
    s = jnp.sum(stats[:, LANES:2 * LANES], axis=1, keepdims=True)
    z = jnp.sum(w * s)
    awb = jnp.sum(w * awp_ref[...], axis=0, keepdims=True) / z
    aw_ref[...] = awb
    part = jnp.dot(awb, x_ref[...], preferred_element_type=jnp.float32)
    ge_ref[...] = jnp.dot(part, wv_ref[...],
                          preferred_element_type=jnp.float32) + bv_ref[...]


def _finish(stats, aw_part, x, Wv, bv2):
    n, d = x.shape
    return pl.pallas_call(
        _fin_body,
        out_shape=[
            jax.ShapeDtypeStruct((1, n), jnp.float32),
            jax.ShapeDtypeStruct((1, d), jnp.float32),
        ],
    )(stats, aw_part, x, Wv, bv2)


def kernel(x, edge_index, Wq, bq, Wk, bk, Wv, bv):
    n, d = x.shape
    e = edge_index.shape[1]
    Wqk = jnp.concatenate([Wq, Wk], axis=1)
    bqk = jnp.concatenate([bq, bk]).reshape(1, 2)
    q2, k2 = _compute_qk(x, Wqk, bqk)
    aw_part, stats = _make_sc_edge(n, e)(
        q2.reshape(n), k2.reshape(n), edge_index.reshape(2 * e))
    aw2, ge2 = _finish(stats.reshape(NC, 2 * LANES), aw_part.reshape(NC, n),
                       x, Wv, bv.reshape(1, d))
    return (ge2.reshape(d), aw2.reshape(n))


# speedup vs baseline: 91.7465x; 1.2128x over previous
import functools

import jax
import jax.numpy as jnp
from jax import lax
from jax.experimental import pallas as pl
from jax.experimental.pallas import tpu as pltpu
from jax.experimental.pallas import tpu_sc as plsc

LANES = 16
NC = 2
NS = 16


def _qk_body(x_ref, wq_ref, bq_ref, wk_ref, bk_ref, q_ref, k_ref):
    xt = x_ref[...].T
    qt = jnp.dot(wq_ref[...].T, xt, preferred_element_type=jnp.float32)
    kt = jnp.dot(wk_ref[...].T, xt, preferred_element_type=jnp.float32)
    q_ref[...] = qt[0] + bq_ref[0]
    k_ref[...] = kt[0] + bk_ref[0]


def _compute_qk(x, Wq, bq, Wk, bk):
    n, d = x.shape
    blk = 1024
    grid = (n + blk - 1) // blk
    return pl.pallas_call(
        _qk_body,
        grid=(grid,),
        in_specs=[
            pl.BlockSpec((blk, d), lambda i: (i, 0)),
            pl.BlockSpec((d, 1), lambda i: (0, 0)),
            pl.BlockSpec((1,), lambda i: (0,)),
            pl.BlockSpec((d, 1), lambda i: (0, 0)),
            pl.BlockSpec((1,), lambda i: (0,)),
        ],
        out_specs=[
            pl.BlockSpec((blk,), lambda i: (i,)),
            pl.BlockSpec((blk,), lambda i: (i,)),
        ],
        out_shape=[
            jax.ShapeDtypeStruct((n,), jnp.float32),
            jax.ShapeDtypeStruct((n,), jnp.float32),
        ],
    )(x, Wq, bq, Wk, bk)


def _make_sc_edge(n, e):
    epw = e // (NC * NS)
    vpi = epw // LANES
    nvec = n // LANES
    stripe = 640
    last_stripe = n - stripe * (NS - 1)

    mesh = plsc.VectorSubcoreMesh(core_axis_name="c", subcore_axis_name="s")

    @functools.partial(
        pl.kernel,
        out_type=(
            jax.ShapeDtypeStruct((n,), jnp.float32),
            jax.ShapeDtypeStruct((n,), jnp.float32),
            jax.ShapeDtypeStruct((NC * 2 * LANES,), jnp.float32),
        ),
        mesh=mesh,
        compiler_params=pltpu.CompilerParams(needs_layout_passes=False),
        scratch_types=[
            pltpu.VMEM((n,), jnp.float32),
            pltpu.VMEM((n,), jnp.float32),
            pltpu.VMEM((epw,), jnp.int32),
            pltpu.VMEM((epw,), jnp.int32),
            pltpu.VMEM((epw,), jnp.float32),
            pltpu.VMEM((n,), jnp.float32),
            pltpu.VMEM((LANES,), jnp.float32),
            pltpu.VMEM((NS * LANES,), jnp.float32),
            pltpu.VMEM((NS * stripe,), jnp.float32),
            pltpu.VMEM((stripe,), jnp.float32),
            pltpu.VMEM((2 * LANES,), jnp.float32),
            pltpu.VMEM_SHARED((NS * LANES,), jnp.float32),
            pltpu.VMEM_SHARED((NS * LANES,), jnp.float32),
            pltpu.VMEM_SHARED((NS * n,), jnp.float32),
            pltpu.SemaphoreType.DMA,
        ],
    )
    def sc_edge(q_hbm, k_hbm, edge_hbm, aw0_hbm, aw1_hbm, stats_hbm,
                q_v, k_v, row_v, col_v, att_v, aw_v,
                red16_v, red_v, stripe_v, out_v, stat_v,
                max_sh, sum_sh, aw_sh, sem):
        c = lax.axis_index("c")
        s = lax.axis_index("s")
        ebase = (c * NS + s) * epw

        cps = [pltpu.async_copy(q_hbm, q_v, sem),
               pltpu.async_copy(k_hbm, k_v, sem),
               pltpu.async_copy(edge_hbm.at[pl.ds(ebase, epw)], row_v, sem),
               pltpu.async_copy(edge_hbm.at[pl.ds(e + ebase, epw)], col_v, sem)]

        zeros = jnp.zeros((LANES,), jnp.float32)

        @plsc.parallel_loop(0, nvec * LANES, LANES, unroll=8)
        def _(i):
            aw_v[pl.ds(i, LANES)] = zeros

        for cp in cps:
            cp.wait()

        @plsc.parallel_loop(0, vpi * LANES, LANES, unroll=4,
                            carry=jnp.full((LANES,), -jnp.inf, jnp.float32))
        def mloc(i, m):
            ridx = row_v[pl.ds(i, LANES)]
            cidx = col_v[pl.ds(i, LANES)]
            a = plsc.load_gather(q_v, [ridx]) * plsc.load_gather(k_v, [cidx])
            a = jnp.where(a >= 0.0, a, a * 0.2)
            att_v[pl.ds(i, LANES)] = a
            return jnp.maximum(m, a)

        red16_v[...] = mloc
        pltpu.sync_copy(red16_v, max_sh.at[pl.ds(s * LANES, LANES)])
        plsc.subcore_barrier()
        pltpu.sync_copy(max_sh, red_v)
        gm = red_v[pl.ds(0, LANES)]
        for t in range(1, NS):
            gm = jnp.maximum(gm, red_v[pl.ds(t * LANES, LANES)])
        m_vec = jnp.full((LANES,), jnp.max(gm), jnp.float32)

        @plsc.parallel_loop(0, vpi * LANES, LANES, unroll=4,
                            carry=jnp.zeros((LANES,), jnp.float32))
        def ssum(i, acc):
            ridx = row_v[pl.ds(i, LANES)]
            p = jnp.exp(att_v[pl.ds(i, LANES)] - m_vec)
            plsc.addupdate_scatter(aw_v, [ridx], p)
            return acc + p

        red16_v[...] = ssum
        pltpu.sync_copy(red16_v, sum_sh.at[pl.ds(s * LANES, LANES)])
        pltpu.sync_copy(aw_v, aw_sh.at[pl.ds(s * n, n)])
        plsc.subcore_barrier()

        def reduce_stripe(base, size):
            scps = [pltpu.async_copy(aw_sh.at[pl.ds(t * n + base, size)],
                                     stripe_v.at[pl.ds(t * stripe, size)], sem)
                    for t in range(NS)]
            for cp in scps:
                cp.wait()

            @plsc.parallel_loop(0, size, LANES, unroll=2)
            def _(j):
                acc = stripe_v[pl.ds(j, LANES)]
                for t in range(1, NS):
                    acc = acc + stripe_v[pl.ds(t * stripe + j, LANES)]
                out_v[pl.ds(j, LANES)] = acc

            @pl.when(c == 0)
            def _():
                pltpu.sync_copy(out_v.at[pl.ds(0, size)],
                                aw0_hbm.at[pl.ds(base, size)])

            @pl.when(c == 1)
            def _():
                pltpu.sync_copy(out_v.at[pl.ds(0, size)],
                                aw1_hbm.at[pl.ds(base, size)])

        @pl.when(s < NS - 1)
        def _():
            reduce_stripe(s * stripe, stripe)

        @pl.when(s == NS - 1)
        def _():
            reduce_stripe((NS - 1) * stripe, last_stripe)

        @pl.when(s == 0)
        def _():
            pltpu.sync_copy(sum_sh, red_v)
            sv = red_v[pl.ds(0, LANES)]
            for t in range(1, NS):
                sv = sv + red_v[pl.ds(t * LANES, LANES)]
            stat_v[pl.ds(0, LANES)] = m_vec
            stat_v[pl.ds(LANES, LANES)] = sv
            pltpu.sync_copy(stat_v, stats_hbm.at[pl.ds(c * 2 * LANES, 2 * LANES)])

    return sc_edge


def _make_fin(n, d, blk):
    def _fin_body(stats_ref, aw0_ref, aw1_ref, x_ref, wv_ref, bv_ref,
                  aw_ref, ge_ref):
        i = pl.program_id(0)
        sv = stats_ref[...]
        m0 = jnp.max(sv[0:LANES])
        m1 = jnp.max(sv[2 * LANES:3 * LANES])
        mg = jnp.maximum(m0, m1)
        w0 = jnp.exp(m0 - mg)
        w1 = jnp.exp(m1 - mg)
        z = w0 * jnp.sum(sv[LANES:2 * LANES]) + w1 * jnp.sum(sv[3 * LANES:])
        awb = (w0 * aw0_ref[...] + w1 * aw1_ref[...]) * (1.0 / z)
        lim = n - i * blk
        lane = lax.broadcasted_iota(jnp.int32, (1, blk), 1)
        awm = jnp.where((lane < lim)[0], awb, 0.0)
        aw_ref[...] = awb
        rows = lax.broadcasted_iota(jnp.int32, (blk, 1), 0)
        xm = jnp.where(rows < lim, x_ref[...], 0.0)
        part = jnp.dot(awm.reshape(1, blk), xm,
                       preferred_element_type=jnp.float32)

        @pl.when(i == 0)
        def _():
            ge_ref[...] = part[0]

        @pl.when(i > 0)
        def _():
            ge_ref[...] = ge_ref[...] + part[0]

        @pl.when(i == pl.num_programs(0) - 1)
        def _():
            ge_ref[...] = jnp.dot(ge_ref[...].reshape(1, d), wv_ref[...],
                                  preferred_element_type=jnp.float32)[0] \
                          + bv_ref[...]

    return _fin_body


def _finish(stats, aw0, aw1, x, Wv, bv):
    n, d = x.shape
    blk = 1024
    grid = (n + blk - 1) // blk
    return pl.pallas_call(
        _make_fin(n, d, blk),
        grid=(grid,),
        in_specs=[
            pl.BlockSpec((4 * LANES,), lambda i: (0,)),
            pl.BlockSpec((blk,), lambda i: (i,)),
            pl.BlockSpec((blk,), lambda i: (i,)),
            pl.BlockSpec((blk, d), lambda i: (i, 0)),
            pl.BlockSpec((d, d), lambda i: (0, 0)),
            pl.BlockSpec((d,), lambda i: (0,)),
        ],
        out_specs=[
            pl.BlockSpec((blk,), lambda i: (i,)),
            pl.BlockSpec((d,), lambda i: (0,)),
        ],
        out_shape=[
            jax.ShapeDtypeStruct((n,), jnp.float32),
            jax.ShapeDtypeStruct((d,), jnp.float32),
        ],
    )(stats, aw0, aw1, x, Wv, bv)


def kernel(x, edge_index, Wq, bq, Wk, bk, Wv, bv):
    n, d = x.shape
    e = edge_index.shape[1]
    q1, k1 = _compute_qk(x, Wq, bq, Wk, bk)
    aw0, aw1, stats = _make_sc_edge(n, e)(q1, k1, edge_index.reshape(2 * e))
    aw, ge = _finish(stats, aw0, aw1, x, Wv, bv)
    return (ge, aw)
